# baseline (speedup 1.0000x reference)
import jax
import jax.numpy as jnp
from jax.experimental import pallas as pl
from jax.experimental.pallas import tpu as pltpu


_B = 4
_S = 2048
_D = 1024
_CH = 1024
_NCK = _S // _CH
_NB = 4
_NITEM = _B * _NCK
_PF = 3


def _body(x_hbm, p_hbm, o_hbm, bufs, pos_v, sem_in, sem_out, sem_p):
    cpp = pltpu.async_copy(p_hbm, pos_v, sem_p)

    def start_in(i):
        b, sc = divmod(i, _NCK)
        return pltpu.async_copy(
            x_hbm.at[b, pl.ds(sc * _CH, _CH)], bufs.at[i % _NB], sem_in.at[i % _NB]
        )

    cps = [None] * _NITEM
    cpo = [None] * _NITEM
    for i in range(_PF):
        cps[i] = start_in(i)
    cpp.wait()

    for i in range(_NITEM):
        b, sc = divmod(i, _NCK)
        k = i % _NB
        cps[i].wait()
        bufs[k] = bufs[k] + pos_v[pl.ds(sc * _CH, _CH), :]
        cpo[i] = pltpu.async_copy(
            bufs.at[k], o_hbm.at[b, pl.ds(sc * _CH, _CH)], sem_out.at[k]
        )
        j = i + _PF
        if j < _NITEM:
            if j >= _NB:
                cpo[j - _NB].wait()
            cps[j] = start_in(j)

    for i in range(_NITEM - _NB, _NITEM):
        cpo[i].wait()


def kernel(inputs, pos_table):
    return pl.pallas_call(
        _body,
        in_specs=[
            pl.BlockSpec(memory_space=pltpu.HBM),
            pl.BlockSpec(memory_space=pltpu.HBM),
        ],
        out_specs=pl.BlockSpec(memory_space=pltpu.HBM),
        out_shape=jax.ShapeDtypeStruct(inputs.shape, inputs.dtype),
        scratch_shapes=[
            pltpu.VMEM((_NB, _CH, _D), jnp.float32),
            pltpu.VMEM((_S, _D), jnp.float32),
            pltpu.SemaphoreType.DMA((_NB,)),
            pltpu.SemaphoreType.DMA((_NB,)),
            pltpu.SemaphoreType.DMA,
        ],
    )(inputs, pos_table)
